# baseline (speedup 1.0000x reference)
import jax
import jax.numpy as jnp
import numpy as np
from jax import lax
from jax.experimental import pallas as pl
from jax.experimental.pallas import tpu as pltpu
from jax.experimental.pallas import tpu_sc as plsc

N_PTS = 1048576
N_OUT = 40
B = 128
SMALL_RES = (16, 32)
SMALL_BASE = (0, 2048)
ST_SIZE = 10240
NSC = 128


def _iota16():
    return lax.iota(jnp.int32, 16)


def _weights(xs, ys, r):
    half = np.float32(0.5 * (r - 1))
    ix = (xs + 1.0) * half
    iy = (ys + 1.0) * half
    ix0 = ix.astype(jnp.int32)
    iy0 = iy.astype(jnp.int32)
    fx = ix - ix0.astype(jnp.float32)
    fy = iy - iy0.astype(jnp.float32)
    wx0 = 1.0 - fx
    wy0 = 1.0 - fy
    return ix0, iy0, wx0 * wy0, fx * wy0, wx0 * fy, fx * fy


def _body(xr, yr, f0, f1, f2, f3, f4, outr, t2o, t3o, t4o, st_sh,
          xb, yb, ix2, ix3, ix4, r2a, r2b, r3a, r3b, r4a, r4b, ob, st,
          sem_a, sem_b, sem_o, sem_p):
    sid = lax.axis_index("s")
    cid = lax.axis_index("c")
    wid = sid * 2 + cid
    sems = (sem_a, sem_b)

    def phase1(buf_s, sum_s, stage4, ssub, tbuf):
        for l, r in enumerate(SMALL_RES):
            fref = (f0, f1)[l]
            rr = r * r
            n = rr // 2
            f_idx = sid // 2
            off = lax.rem(sid, 2) * n
            ds = []
            for cc in range(4):
                ds.append(pltpu.async_copy(
                    fref.at[cc, f_idx, pl.ds(off, n)],
                    buf_s.at[cc, pl.ds(0, n)], sem_p))
            for d in ds:
                d.wait()

            @pl.loop(0, n // 16)
            def _sum_small(k):
                o = k * 16
                v = (buf_s[0, pl.ds(o, 16)] + buf_s[1, pl.ds(o, 16)]
                     + buf_s[2, pl.ds(o, 16)] + buf_s[3, pl.ds(o, 16)])
                sum_s[pl.ds(o, 16)] = v

            pltpu.sync_copy(
                sum_s.at[pl.ds(0, n)],
                st_sh.at[pl.ds(SMALL_BASE[l] + f_idx * rr + off, n)])

        for (r, fref, tsh) in ((64, f2, t2o), (128, f3, t3o),
                               (256, f4, t4o)):
            rr = r * r
            rows_per_tile = rr // 16
            ns = min(NSC, rows_per_tile)
            pat = (_iota16() & 7) * (ns + 17) + (_iota16() >> 3)

            @pl.loop(0, rows_per_tile // ns)
            def _subchunk(si):
                sbase = sid * rows_per_tile + si * ns
                dbase = cid * rr + sbase
                toff = jnp.minimum(sbase + ns, rr - 16)
                ds = []
                for cc in range(4):
                    ds.append(pltpu.async_copy(
                        fref.at[cc, :, pl.ds(sbase, ns)],
                        stage4.at[cc, :, pl.ds(0, ns)], sem_p))
                    ds.append(pltpu.async_copy(
                        fref.at[cc, :, pl.ds(toff, 16)],
                        stage4.at[cc, :, pl.ds(ns, 16)], sem_p))
                for d in ds:
                    d.wait()
                for f in range(8):
                    @pl.loop(0, (ns + 16) // 16)
                    def _sum_big(k):
                        o = k * 16
                        v = (stage4[0, f, pl.ds(o, 16)]
                             + stage4[1, f, pl.ds(o, 16)]
                             + stage4[2, f, pl.ds(o, 16)]
                             + stage4[3, f, pl.ds(o, 16)])
                        ssub[pl.ds(f * (ns + 17) + o, 16)] = v

                @pl.loop(0, ns)
                def _build_t(p):
                    tbuf[p, :] = plsc.load_gather(ssub, [pat + p])

                pltpu.sync_copy(tbuf.at[pl.ds(0, ns), :],
                                tsh.at[pl.ds(dbase, ns), :])

    pl.run_scoped(
        phase1,
        pltpu.VMEM((4, 512), jnp.float32),
        pltpu.VMEM((512,), jnp.float32),
        pltpu.VMEM((4, 8, NSC + 16), jnp.float32),
        pltpu.VMEM((8 * (NSC + 17),), jnp.float32),
        pltpu.VMEM((NSC, 16), jnp.float32),
    )

    plsc.subcore_barrier()

    def phase2():
        pw = N_PTS // 32
        nch = pw // B
        pltpu.sync_copy(st_sh, st)
        stream_cfg = ((64, ix2, r2a, r2b, t2o),
                      (128, ix3, r3a, r3b, t3o),
                      (256, ix4, r4a, r4b, t4o))

        def do_prefetch(ch, par):
            base = wid * pw + ch * B
            ds = []
            for j in range(B // 128):
                ds.append(pltpu.async_copy(
                    xr.at[pl.ds(base + j * 128, 128)], xb.at[par, j], sem_p))
                ds.append(pltpu.async_copy(
                    yr.at[pl.ds(base + j * 128, 128)], yb.at[par, j], sem_p))
            for d in ds:
                d.wait()

            @pl.loop(0, B // 128)
            def _pass_a(j):
                @pl.loop(0, 8)
                def _grp(gg):
                    o = gg * 16
                    xs = xb[par, j, pl.ds(o, 16)] * 2.0 - 1.0
                    ys = yb[par, j, pl.ds(o, 16)] * 2.0 - 1.0
                    for (r, ix_, _, _, _) in stream_cfg:
                        half = np.float32(0.5 * (r - 1))
                        coff = cid * (r * r)
                        ix0 = ((xs + 1.0) * half).astype(jnp.int32)
                        iy0 = ((ys + 1.0) * half).astype(jnp.int32)
                        iy1 = jnp.minimum(iy0 + 1, r - 1)
                        b0 = iy0 * r + ix0 + coff
                        ix_[par, 0, j, pl.ds(o, 16)] = b0
                        ix_[par, 1, j, pl.ds(o, 16)] = (
                            b0 + (jnp.minimum(iy0 + 1, r - 1) - iy0) * r)

            for j in range(B // 128):
                s = pl.ds(j * 128, 128)
                sem = sems[par]
                for (_, ix_, ra_, rb_, tsh) in stream_cfg:
                    pltpu.async_copy(tsh.at[ix_.at[par, 0, j]],
                                     ra_.at[par, s, :], sem)
                    pltpu.async_copy(tsh.at[ix_.at[par, 1, j]],
                                     rb_.at[par, s, :], sem)
                del tsh

        def drain_rows(par):
            for rows in (r2a, r2b, r3a, r3b, r4a, r4b):
                pltpu.make_async_copy(
                    outr.at[pl.ds(0, B), pl.ds(0, 16)],
                    rows.at[par], sems[par]).wait()

        def blend(par):
            @pl.loop(0, B // 128)
            def _blk(j):
                @pl.loop(0, 8)
                def _grp(gg):
                    o = gg * 16
                    xs = xb[par, j, pl.ds(o, 16)] * 2.0 - 1.0
                    ys = yb[par, j, pl.ds(o, 16)] * 2.0 - 1.0
                    pvec = j * 128 + o + _iota16()
                    for l, r in enumerate(SMALL_RES):
                        rr = r * r
                        ix0, iy0, w00, w01, w10, w11 = _weights(xs, ys, r)
                        ix1 = jnp.minimum(ix0 + 1, r - 1)
                        iy1 = jnp.minimum(iy0 + 1, r - 1)
                        b00 = iy0 * r + ix0
                        dx = ix1 - ix0
                        b01 = b00 + dx
                        b10 = iy1 * r + ix0
                        b11 = b10 + dx
                        for f in range(8):
                            kf = (f + _iota16()) & 7
                            cf = SMALL_BASE[l] + kf * rr
                            v00 = plsc.load_gather(st, [b00 + cf])
                            v01 = plsc.load_gather(st, [b01 + cf])
                            v10 = plsc.load_gather(st, [b10 + cf])
                            v11 = plsc.load_gather(st, [b11 + cf])
                            of = (v00 * w00 + v01 * w01
                                  + v10 * w10 + v11 * w11)
                            plsc.store_scatter(
                                ob.at[par], [pvec, kf + l * 8], of)
                    for li, (r, _, ra_, rb_, _) in enumerate(stream_cfg):
                        _, _, w00, w01, w10, w11 = _weights(xs, ys, r)
                        for f in range(8):
                            kf = (f + _iota16()) & 7
                            kf8 = kf + 8
                            v00 = plsc.load_gather(ra_.at[par], [pvec, kf])
                            v01 = plsc.load_gather(ra_.at[par], [pvec, kf8])
                            v10 = plsc.load_gather(rb_.at[par], [pvec, kf])
                            v11 = plsc.load_gather(rb_.at[par], [pvec, kf8])
                            of = (v00 * w00 + v01 * w01
                                  + v10 * w10 + v11 * w11)
                            plsc.store_scatter(
                                ob.at[par],
                                [pvec, kf + (16 + li * 8)], of)

        do_prefetch(0, 0)

        @pl.loop(0, nch, step=2)
        def _chunk2(ch0):
            for sub in range(2):
                ch = ch0 + sub

                @pl.when(ch + 1 < nch)
                def _():
                    do_prefetch(ch + 1, 1 - sub)

                drain_rows(sub)

                @pl.when(ch >= 2)
                def _():
                    pltpu.make_async_copy(
                        outr.at[pl.ds(0, B), :], ob.at[sub], sem_o).wait()

                blend(sub)
                base = wid * pw + ch * B
                pltpu.async_copy(ob.at[sub], outr.at[pl.ds(base, B), :],
                                 sem_o)

        for par in range(2):
            pltpu.make_async_copy(
                outr.at[pl.ds(0, B), :], ob.at[par], sem_o).wait()

    phase2()


def kernel(x, y, F0, F1, F2, F3, F4):
    f32 = jnp.float32
    mesh = plsc.VectorSubcoreMesh(core_axis_name="c", subcore_axis_name="s")
    cparams = pltpu.CompilerParams(needs_layout_passes=False,
                                   use_tc_tiling_on_sc=False)

    main = pl.kernel(
        _body,
        out_type=(
            jax.ShapeDtypeStruct((N_PTS, N_OUT), f32),
            jax.ShapeDtypeStruct((2 * 64 * 64, 16), f32),
            jax.ShapeDtypeStruct((2 * 128 * 128, 16), f32),
            jax.ShapeDtypeStruct((2 * 256 * 256, 16), f32),
        ),
        mesh=mesh,
        scratch_types=[
            pltpu.VMEM_SHARED((ST_SIZE,), f32),
            pltpu.VMEM((2, B // 128, 128), f32),
            pltpu.VMEM((2, B // 128, 128), f32),
            pltpu.VMEM((2, 2, B // 128, 128), jnp.int32),
            pltpu.VMEM((2, 2, B // 128, 128), jnp.int32),
            pltpu.VMEM((2, 2, B // 128, 128), jnp.int32),
            pltpu.VMEM((2, B, 16), f32),
            pltpu.VMEM((2, B, 16), f32),
            pltpu.VMEM((2, B, 16), f32),
            pltpu.VMEM((2, B, 16), f32),
            pltpu.VMEM((2, B, 16), f32),
            pltpu.VMEM((2, B, 16), f32),
            pltpu.VMEM((2, B, N_OUT), f32),
            pltpu.VMEM((ST_SIZE,), f32),
            pltpu.SemaphoreType.DMA,
            pltpu.SemaphoreType.DMA,
            pltpu.SemaphoreType.DMA,
            pltpu.SemaphoreType.DMA,
        ],
        compiler_params=cparams,
    )

    fs = [F.reshape(4, 8, r * r)
          for F, r in zip((F0, F1, F2, F3, F4), (16, 32, 64, 128, 256))]
    return main(x.reshape(-1), y.reshape(-1), *fs)[0]


# speedup vs baseline: 1.0727x; 1.0727x over previous
import jax
import jax.numpy as jnp
import numpy as np
from jax import lax
from jax.experimental import pallas as pl
from jax.experimental.pallas import tpu as pltpu
from jax.experimental.pallas import tpu_sc as plsc

N_PTS = 1048576
N_OUT = 40
B = 128
SMALL_RES = (16, 32)
SMALL_BASE = (0, 2048)
ST_SIZE = 10240
NSC = 128


def _iota16():
    return lax.iota(jnp.int32, 16)


def _weights(xs, ys, r):
    half = np.float32(0.5 * (r - 1))
    ix = (xs + 1.0) * half
    iy = (ys + 1.0) * half
    ix0 = ix.astype(jnp.int32)
    iy0 = iy.astype(jnp.int32)
    fx = ix - ix0.astype(jnp.float32)
    fy = iy - iy0.astype(jnp.float32)
    wx0 = 1.0 - fx
    wy0 = 1.0 - fy
    return ix0, iy0, wx0 * wy0, fx * wy0, wx0 * fy, fx * fy


def _body(xr, yr, f0, f1, f2, f3, f4, outr, t2o, t3o, t4o, st_sh,
          xb, yb, ix2, ix3, ix4, r2a, r2b, r3a, r3b, r4a, r4b, ob, st,
          sem_a, sem_b, sem_o, sem_p):
    sid = lax.axis_index("s")
    cid = lax.axis_index("c")
    wid = sid * 2 + cid
    sems = (sem_a, sem_b)

    def phase1(buf_s, sum_s, stage4, ssub, tbuf):
        for l, r in enumerate(SMALL_RES):
            fref = (f0, f1)[l]
            rr = r * r
            n = rr // 2
            f_idx = sid // 2
            off = lax.rem(sid, 2) * n
            toff = jnp.minimum(off + n, rr - 16)
            ds = []
            for cc in range(4):
                ds.append(pltpu.async_copy(
                    fref.at[cc, f_idx, pl.ds(off, n)],
                    buf_s.at[cc, pl.ds(0, n)], sem_p))
                ds.append(pltpu.async_copy(
                    fref.at[cc, f_idx, pl.ds(toff, 16)],
                    buf_s.at[cc, pl.ds(n, 16)], sem_p))
            for d in ds:
                d.wait()

            @pl.loop(0, (n + 16) // 16)
            def _sum_small(k):
                o = k * 16
                v = (buf_s[0, pl.ds(o, 16)] + buf_s[1, pl.ds(o, 16)]
                     + buf_s[2, pl.ds(o, 16)] + buf_s[3, pl.ds(o, 16)])
                sum_s[pl.ds(o, 16)] = v

            @pl.loop(0, n // 16)
            def _pack_small(k):
                o = k * 16
                a = sum_s[pl.ds(o, 16)]
                b = sum_s[pl.ds(o + 1, 16)]
                pk = plsc.bitcast(
                    plsc.pack(a, b, format=plsc.PackFormat.INTERLEAVED),
                    jnp.float32)
                sum_s[pl.ds(o, 16)] = pk

            pltpu.sync_copy(
                sum_s.at[pl.ds(0, n)],
                st_sh.at[pl.ds(SMALL_BASE[l] + f_idx * rr + off, n)])

        for (r, fref, tsh) in ((64, f2, t2o), (128, f3, t3o),
                               (256, f4, t4o)):
            rr = r * r
            rows_per_tile = rr // 16
            ns = min(NSC, rows_per_tile)
            pat_a = (_iota16() & 7) * (ns + 17)

            @pl.loop(0, rows_per_tile // ns)
            def _subchunk(si):
                sbase = sid * rows_per_tile + si * ns
                dbase = cid * rr + sbase
                toff = jnp.minimum(sbase + ns, rr - 16)
                ds = []
                for cc in range(4):
                    ds.append(pltpu.async_copy(
                        fref.at[cc, :, pl.ds(sbase, ns)],
                        stage4.at[cc, :, pl.ds(0, ns)], sem_p))
                    ds.append(pltpu.async_copy(
                        fref.at[cc, :, pl.ds(toff, 16)],
                        stage4.at[cc, :, pl.ds(ns, 16)], sem_p))
                for d in ds:
                    d.wait()
                for f in range(8):
                    @pl.loop(0, (ns + 16) // 16)
                    def _sum_big(k):
                        o = k * 16
                        v = (stage4[0, f, pl.ds(o, 16)]
                             + stage4[1, f, pl.ds(o, 16)]
                             + stage4[2, f, pl.ds(o, 16)]
                             + stage4[3, f, pl.ds(o, 16)])
                        ssub[pl.ds(f * (ns + 17) + o, 16)] = v

                @pl.loop(0, ns)
                def _build_t(p):
                    va = plsc.load_gather(ssub, [pat_a + p])
                    vb = plsc.load_gather(ssub, [pat_a + (p + 1)])
                    tbuf[p, :] = plsc.bitcast(
                        plsc.pack(va, vb,
                                  format=plsc.PackFormat.INTERLEAVED),
                        jnp.float32)

                pltpu.sync_copy(tbuf.at[pl.ds(0, ns), :],
                                tsh.at[pl.ds(dbase, ns), :])

    pl.run_scoped(
        phase1,
        pltpu.VMEM((4, 528), jnp.float32),
        pltpu.VMEM((528,), jnp.float32),
        pltpu.VMEM((4, 8, NSC + 16), jnp.float32),
        pltpu.VMEM((8 * (NSC + 17),), jnp.float32),
        pltpu.VMEM((NSC, 16), jnp.float32),
    )

    plsc.subcore_barrier()

    def phase2():
        pw = N_PTS // 32
        nch = pw // B
        pltpu.sync_copy(st_sh, st)
        stream_cfg = ((64, ix2, r2a, r2b, t2o),
                      (128, ix3, r3a, r3b, t3o),
                      (256, ix4, r4a, r4b, t4o))

        def do_prefetch(ch, par):
            base = wid * pw + ch * B
            ds = []
            for j in range(B // 128):
                ds.append(pltpu.async_copy(
                    xr.at[pl.ds(base + j * 128, 128)], xb.at[par, j], sem_p))
                ds.append(pltpu.async_copy(
                    yr.at[pl.ds(base + j * 128, 128)], yb.at[par, j], sem_p))
            for d in ds:
                d.wait()

            @pl.loop(0, B // 128)
            def _pass_a(j):
                @pl.loop(0, 8)
                def _grp(gg):
                    o = gg * 16
                    xs = xb[par, j, pl.ds(o, 16)] * 2.0 - 1.0
                    ys = yb[par, j, pl.ds(o, 16)] * 2.0 - 1.0
                    for (r, ix_, _, _, _) in stream_cfg:
                        half = np.float32(0.5 * (r - 1))
                        coff = cid * (r * r)
                        ix0 = ((xs + 1.0) * half).astype(jnp.int32)
                        iy0 = ((ys + 1.0) * half).astype(jnp.int32)
                        iy1 = jnp.minimum(iy0 + 1, r - 1)
                        b0 = iy0 * r + ix0 + coff
                        ix_[par, 0, j, pl.ds(o, 16)] = b0
                        ix_[par, 1, j, pl.ds(o, 16)] = (
                            b0 + (jnp.minimum(iy0 + 1, r - 1) - iy0) * r)

            for j in range(B // 128):
                s = pl.ds(j * 128, 128)
                sem = sems[par]
                for (_, ix_, ra_, rb_, tsh) in stream_cfg:
                    pltpu.async_copy(tsh.at[ix_.at[par, 0, j]],
                                     ra_.at[par, s, :], sem)
                    pltpu.async_copy(tsh.at[ix_.at[par, 1, j]],
                                     rb_.at[par, s, :], sem)
                del tsh

        def drain_rows(par):
            for rows in (r2a, r2b, r3a, r3b, r4a, r4b):
                pltpu.make_async_copy(
                    outr.at[pl.ds(0, B), pl.ds(0, 16)],
                    rows.at[par], sems[par]).wait()

        def blend(par):
            @pl.loop(0, B // 128)
            def _blk(j):
                @pl.loop(0, 8)
                def _grp(gg):
                    o = gg * 16
                    xs = xb[par, j, pl.ds(o, 16)] * 2.0 - 1.0
                    ys = yb[par, j, pl.ds(o, 16)] * 2.0 - 1.0
                    pvec = j * 128 + o + _iota16()
                    fmt = plsc.PackFormat.INTERLEAVED
                    for l, r in enumerate(SMALL_RES):
                        rr = r * r
                        ix0, iy0, w00, w01, w10, w11 = _weights(xs, ys, r)
                        iy1 = jnp.minimum(iy0 + 1, r - 1)
                        b00 = iy0 * r + ix0
                        b10 = iy1 * r + ix0
                        for f in range(8):
                            cf = SMALL_BASE[l] + f * rr
                            wa = plsc.load_gather(st, [b00 + cf])
                            wb = plsc.load_gather(st, [b10 + cf])
                            v00, v01 = plsc.unpack(
                                plsc.bitcast(wa, jnp.bfloat16), format=fmt)
                            v10, v11 = plsc.unpack(
                                plsc.bitcast(wb, jnp.bfloat16), format=fmt)
                            of = (v00 * w00 + v01 * w01
                                  + v10 * w10 + v11 * w11)
                            plsc.store_scatter(
                                ob.at[par],
                                [pvec, jnp.full((16,), l * 8 + f,
                                                jnp.int32)], of)
                    for li, (r, _, ra_, rb_, _) in enumerate(stream_cfg):
                        _, _, w00, w01, w10, w11 = _weights(xs, ys, r)
                        for f in range(8):
                            kf = (f + _iota16()) & 15
                            wa = plsc.load_gather(ra_.at[par], [pvec, kf])
                            wb = plsc.load_gather(rb_.at[par], [pvec, kf])
                            v00, v01 = plsc.unpack(
                                plsc.bitcast(wa, jnp.bfloat16), format=fmt)
                            v10, v11 = plsc.unpack(
                                plsc.bitcast(wb, jnp.bfloat16), format=fmt)
                            of = (v00 * w00 + v01 * w01
                                  + v10 * w10 + v11 * w11)
                            plsc.store_scatter(
                                ob.at[par],
                                [pvec, (kf & 7) + (16 + li * 8)], of)

        do_prefetch(0, 0)

        @pl.loop(0, nch, step=2)
        def _chunk2(ch0):
            for sub in range(2):
                ch = ch0 + sub

                @pl.when(ch + 1 < nch)
                def _():
                    do_prefetch(ch + 1, 1 - sub)

                drain_rows(sub)

                @pl.when(ch >= 2)
                def _():
                    pltpu.make_async_copy(
                        outr.at[pl.ds(0, B), :], ob.at[sub], sem_o).wait()

                blend(sub)
                base = wid * pw + ch * B
                pltpu.async_copy(ob.at[sub], outr.at[pl.ds(base, B), :],
                                 sem_o)

        for par in range(2):
            pltpu.make_async_copy(
                outr.at[pl.ds(0, B), :], ob.at[par], sem_o).wait()

    phase2()


def kernel(x, y, F0, F1, F2, F3, F4):
    f32 = jnp.float32
    mesh = plsc.VectorSubcoreMesh(core_axis_name="c", subcore_axis_name="s")
    cparams = pltpu.CompilerParams(needs_layout_passes=False,
                                   use_tc_tiling_on_sc=False)

    main = pl.kernel(
        _body,
        out_type=(
            jax.ShapeDtypeStruct((N_PTS, N_OUT), f32),
            jax.ShapeDtypeStruct((2 * 64 * 64, 16), f32),
            jax.ShapeDtypeStruct((2 * 128 * 128, 16), f32),
            jax.ShapeDtypeStruct((2 * 256 * 256, 16), f32),
        ),
        mesh=mesh,
        scratch_types=[
            pltpu.VMEM_SHARED((ST_SIZE,), f32),
            pltpu.VMEM((2, B // 128, 128), f32),
            pltpu.VMEM((2, B // 128, 128), f32),
            pltpu.VMEM((2, 2, B // 128, 128), jnp.int32),
            pltpu.VMEM((2, 2, B // 128, 128), jnp.int32),
            pltpu.VMEM((2, 2, B // 128, 128), jnp.int32),
            pltpu.VMEM((2, B, 16), f32),
            pltpu.VMEM((2, B, 16), f32),
            pltpu.VMEM((2, B, 16), f32),
            pltpu.VMEM((2, B, 16), f32),
            pltpu.VMEM((2, B, 16), f32),
            pltpu.VMEM((2, B, 16), f32),
            pltpu.VMEM((2, B, N_OUT), f32),
            pltpu.VMEM((ST_SIZE,), f32),
            pltpu.SemaphoreType.DMA,
            pltpu.SemaphoreType.DMA,
            pltpu.SemaphoreType.DMA,
            pltpu.SemaphoreType.DMA,
        ],
        compiler_params=cparams,
    )

    fs = [F.reshape(4, 8, r * r)
          for F, r in zip((F0, F1, F2, F3, F4), (16, 32, 64, 128, 256))]
    return main(x.reshape(-1), y.reshape(-1), *fs)[0]


# speedup vs baseline: 1.1010x; 1.0264x over previous
import jax
import jax.numpy as jnp
import numpy as np
from jax import lax
from jax.experimental import pallas as pl
from jax.experimental.pallas import tpu as pltpu
from jax.experimental.pallas import tpu_sc as plsc

N_PTS = 1048576
N_OUT = 40
B = 256
SMALL_RES = (16, 32)
SMALL_BASE = (0, 2048)
ST_SIZE = 10240
NSC = 128


def _iota16():
    return lax.iota(jnp.int32, 16)


def _weights(xs, ys, r):
    half = np.float32(0.5 * (r - 1))
    ix = (xs + 1.0) * half
    iy = (ys + 1.0) * half
    ix0 = ix.astype(jnp.int32)
    iy0 = iy.astype(jnp.int32)
    fx = ix - ix0.astype(jnp.float32)
    fy = iy - iy0.astype(jnp.float32)
    wx0 = 1.0 - fx
    wy0 = 1.0 - fy
    return ix0, iy0, wx0 * wy0, fx * wy0, wx0 * fy, fx * fy


def _body(xr, yr, f0, f1, f2, f3, f4, outr, t2o, t3o, t4o, st_sh,
          xb, yb, ix2, ix3, ix4, r2a, r2b, r3a, r3b, r4a, r4b, ob, st,
          sem_a, sem_b, sem_o, sem_p):
    sid = lax.axis_index("s")
    cid = lax.axis_index("c")
    wid = sid * 2 + cid
    sems = (sem_a, sem_b)

    def phase1(buf_s, sum_s, stage4, ssub, tbuf):
        for l, r in enumerate(SMALL_RES):
            fref = (f0, f1)[l]
            rr = r * r
            n = rr // 2
            f_idx = sid // 2
            off = lax.rem(sid, 2) * n
            toff = jnp.minimum(off + n, rr - 16)
            ds = []
            for cc in range(4):
                ds.append(pltpu.async_copy(
                    fref.at[cc, f_idx, pl.ds(off, n)],
                    buf_s.at[cc, pl.ds(0, n)], sem_p))
                ds.append(pltpu.async_copy(
                    fref.at[cc, f_idx, pl.ds(toff, 16)],
                    buf_s.at[cc, pl.ds(n, 16)], sem_p))
            for d in ds:
                d.wait()

            @pl.loop(0, (n + 16) // 16)
            def _sum_small(k):
                o = k * 16
                v = (buf_s[0, pl.ds(o, 16)] + buf_s[1, pl.ds(o, 16)]
                     + buf_s[2, pl.ds(o, 16)] + buf_s[3, pl.ds(o, 16)])
                sum_s[pl.ds(o, 16)] = v

            @pl.loop(0, n // 16)
            def _pack_small(k):
                o = k * 16
                a = sum_s[pl.ds(o, 16)]
                b = sum_s[pl.ds(o + 1, 16)]
                pk = plsc.bitcast(
                    plsc.pack(a, b, format=plsc.PackFormat.INTERLEAVED),
                    jnp.float32)
                sum_s[pl.ds(o, 16)] = pk

            pltpu.sync_copy(
                sum_s.at[pl.ds(0, n)],
                st_sh.at[pl.ds(SMALL_BASE[l] + f_idx * rr + off, n)])

        for (r, fref, tsh) in ((64, f2, t2o), (128, f3, t3o),
                               (256, f4, t4o)):
            rr = r * r
            rows_per_tile = rr // 16
            ns = min(NSC, rows_per_tile)
            pat_a = (_iota16() & 7) * (ns + 17)

            @pl.loop(0, rows_per_tile // ns)
            def _subchunk(si):
                sbase = sid * rows_per_tile + si * ns
                dbase = cid * rr + sbase
                toff = jnp.minimum(sbase + ns, rr - 16)
                ds = []
                for cc in range(4):
                    ds.append(pltpu.async_copy(
                        fref.at[cc, :, pl.ds(sbase, ns)],
                        stage4.at[cc, :, pl.ds(0, ns)], sem_p))
                    ds.append(pltpu.async_copy(
                        fref.at[cc, :, pl.ds(toff, 16)],
                        stage4.at[cc, :, pl.ds(ns, 16)], sem_p))
                for d in ds:
                    d.wait()
                for f in range(8):
                    @pl.loop(0, (ns + 16) // 16)
                    def _sum_big(k):
                        o = k * 16
                        v = (stage4[0, f, pl.ds(o, 16)]
                             + stage4[1, f, pl.ds(o, 16)]
                             + stage4[2, f, pl.ds(o, 16)]
                             + stage4[3, f, pl.ds(o, 16)])
                        ssub[pl.ds(f * (ns + 17) + o, 16)] = v

                @pl.loop(0, ns)
                def _build_t(p):
                    va = plsc.load_gather(ssub, [pat_a + p])
                    vb = plsc.load_gather(ssub, [pat_a + (p + 1)])
                    tbuf[p, :] = plsc.bitcast(
                        plsc.pack(va, vb,
                                  format=plsc.PackFormat.INTERLEAVED),
                        jnp.float32)

                pltpu.sync_copy(tbuf.at[pl.ds(0, ns), :],
                                tsh.at[pl.ds(dbase, ns), :])

    pl.run_scoped(
        phase1,
        pltpu.VMEM((4, 528), jnp.float32),
        pltpu.VMEM((528,), jnp.float32),
        pltpu.VMEM((4, 8, NSC + 16), jnp.float32),
        pltpu.VMEM((8 * (NSC + 17),), jnp.float32),
        pltpu.VMEM((NSC, 16), jnp.float32),
    )

    plsc.subcore_barrier()

    def phase2():
        pw = N_PTS // 32
        nch = pw // B
        pltpu.sync_copy(st_sh, st)
        stream_cfg = ((64, ix2, r2a, r2b, t2o),
                      (128, ix3, r3a, r3b, t3o),
                      (256, ix4, r4a, r4b, t4o))

        def do_prefetch(ch, par):
            base = wid * pw + ch * B
            ds = []
            for j in range(B // 128):
                ds.append(pltpu.async_copy(
                    xr.at[pl.ds(base + j * 128, 128)], xb.at[par, j], sem_p))
                ds.append(pltpu.async_copy(
                    yr.at[pl.ds(base + j * 128, 128)], yb.at[par, j], sem_p))
            for d in ds:
                d.wait()

            @pl.loop(0, B // 128)
            def _pass_a(j):
                @pl.loop(0, 8)
                def _grp(gg):
                    o = gg * 16
                    xs = xb[par, j, pl.ds(o, 16)] * 2.0 - 1.0
                    ys = yb[par, j, pl.ds(o, 16)] * 2.0 - 1.0
                    for (r, ix_, _, _, _) in stream_cfg:
                        half = np.float32(0.5 * (r - 1))
                        coff = cid * (r * r)
                        ix0 = ((xs + 1.0) * half).astype(jnp.int32)
                        iy0 = ((ys + 1.0) * half).astype(jnp.int32)
                        iy1 = jnp.minimum(iy0 + 1, r - 1)
                        b0 = iy0 * r + ix0 + coff
                        ix_[par, 0, j, pl.ds(o, 16)] = b0
                        ix_[par, 1, j, pl.ds(o, 16)] = (
                            b0 + (jnp.minimum(iy0 + 1, r - 1) - iy0) * r)

            for j in range(B // 128):
                s = pl.ds(j * 128, 128)
                sem = sems[par]
                for (_, ix_, ra_, rb_, tsh) in stream_cfg:
                    pltpu.async_copy(tsh.at[ix_.at[par, 0, j]],
                                     ra_.at[par, s, :], sem)
                    pltpu.async_copy(tsh.at[ix_.at[par, 1, j]],
                                     rb_.at[par, s, :], sem)
                del tsh

        def drain_rows(par):
            for rows in (r2a, r2b, r3a, r3b, r4a, r4b):
                pltpu.make_async_copy(
                    outr.at[pl.ds(0, B), pl.ds(0, 16)],
                    rows.at[par], sems[par]).wait()

        def blend(par):
            @pl.loop(0, B // 128)
            def _blk(j):
                @pl.loop(0, 8)
                def _grp(gg):
                    o = gg * 16
                    xs = xb[par, j, pl.ds(o, 16)] * 2.0 - 1.0
                    ys = yb[par, j, pl.ds(o, 16)] * 2.0 - 1.0
                    pvec = j * 128 + o + _iota16()
                    fmt = plsc.PackFormat.INTERLEAVED
                    for l, r in enumerate(SMALL_RES):
                        rr = r * r
                        ix0, iy0, w00, w01, w10, w11 = _weights(xs, ys, r)
                        iy1 = jnp.minimum(iy0 + 1, r - 1)
                        b00 = iy0 * r + ix0
                        b10 = iy1 * r + ix0
                        for f in range(8):
                            cf = SMALL_BASE[l] + f * rr
                            wa = plsc.load_gather(st, [b00 + cf])
                            wb = plsc.load_gather(st, [b10 + cf])
                            v00, v01 = plsc.unpack(
                                plsc.bitcast(wa, jnp.bfloat16), format=fmt)
                            v10, v11 = plsc.unpack(
                                plsc.bitcast(wb, jnp.bfloat16), format=fmt)
                            of = (v00 * w00 + v01 * w01
                                  + v10 * w10 + v11 * w11)
                            plsc.store_scatter(
                                ob.at[par],
                                [pvec, jnp.full((16,), l * 8 + f,
                                                jnp.int32)], of)
                    for li, (r, _, ra_, rb_, _) in enumerate(stream_cfg):
                        _, _, w00, w01, w10, w11 = _weights(xs, ys, r)
                        for f in range(8):
                            kf = (f + _iota16()) & 15
                            wa = plsc.load_gather(ra_.at[par], [pvec, kf])
                            wb = plsc.load_gather(rb_.at[par], [pvec, kf])
                            v00, v01 = plsc.unpack(
                                plsc.bitcast(wa, jnp.bfloat16), format=fmt)
                            v10, v11 = plsc.unpack(
                                plsc.bitcast(wb, jnp.bfloat16), format=fmt)
                            of = (v00 * w00 + v01 * w01
                                  + v10 * w10 + v11 * w11)
                            plsc.store_scatter(
                                ob.at[par],
                                [pvec, (kf & 7) + (16 + li * 8)], of)

        do_prefetch(0, 0)

        @pl.loop(0, nch, step=2)
        def _chunk2(ch0):
            for sub in range(2):
                ch = ch0 + sub

                @pl.when(ch + 1 < nch)
                def _():
                    do_prefetch(ch + 1, 1 - sub)

                drain_rows(sub)

                @pl.when(ch >= 2)
                def _():
                    pltpu.make_async_copy(
                        outr.at[pl.ds(0, B), :], ob.at[sub], sem_o).wait()

                blend(sub)
                base = wid * pw + ch * B
                pltpu.async_copy(ob.at[sub], outr.at[pl.ds(base, B), :],
                                 sem_o)

        for par in range(2):
            pltpu.make_async_copy(
                outr.at[pl.ds(0, B), :], ob.at[par], sem_o).wait()

    phase2()


def kernel(x, y, F0, F1, F2, F3, F4):
    f32 = jnp.float32
    mesh = plsc.VectorSubcoreMesh(core_axis_name="c", subcore_axis_name="s")
    cparams = pltpu.CompilerParams(needs_layout_passes=False,
                                   use_tc_tiling_on_sc=False)

    main = pl.kernel(
        _body,
        out_type=(
            jax.ShapeDtypeStruct((N_PTS, N_OUT), f32),
            jax.ShapeDtypeStruct((2 * 64 * 64, 16), f32),
            jax.ShapeDtypeStruct((2 * 128 * 128, 16), f32),
            jax.ShapeDtypeStruct((2 * 256 * 256, 16), f32),
        ),
        mesh=mesh,
        scratch_types=[
            pltpu.VMEM_SHARED((ST_SIZE,), f32),
            pltpu.VMEM((2, B // 128, 128), f32),
            pltpu.VMEM((2, B // 128, 128), f32),
            pltpu.VMEM((2, 2, B // 128, 128), jnp.int32),
            pltpu.VMEM((2, 2, B // 128, 128), jnp.int32),
            pltpu.VMEM((2, 2, B // 128, 128), jnp.int32),
            pltpu.VMEM((2, B, 16), f32),
            pltpu.VMEM((2, B, 16), f32),
            pltpu.VMEM((2, B, 16), f32),
            pltpu.VMEM((2, B, 16), f32),
            pltpu.VMEM((2, B, 16), f32),
            pltpu.VMEM((2, B, 16), f32),
            pltpu.VMEM((2, B, N_OUT), f32),
            pltpu.VMEM((ST_SIZE,), f32),
            pltpu.SemaphoreType.DMA,
            pltpu.SemaphoreType.DMA,
            pltpu.SemaphoreType.DMA,
            pltpu.SemaphoreType.DMA,
        ],
        compiler_params=cparams,
    )

    fs = [F.reshape(4, 8, r * r)
          for F, r in zip((F0, F1, F2, F3, F4), (16, 32, 64, 128, 256))]
    return main(x.reshape(-1), y.reshape(-1), *fs)[0]
